# baseline (speedup 1.0000x reference)
import jax
import jax.numpy as jnp
from jax.experimental import pallas as pl
from jax.experimental.pallas import tpu as pltpu

_R = 512
_NBUF = 6


def _ring_body(x_hbm, p_hbm, o_hbm, xb, ob, pv, sx, sp, so):
    n_rows = x_hbm.shape[0]
    seq_len = p_hbm.shape[0]
    n_blocks = n_rows // _R
    n_pos_chunks = seq_len // _R

    for k in range(n_pos_chunks):
        pltpu.async_copy(
            p_hbm.at[pl.ds(k * _R, _R)], pv.at[pl.ds(k * _R, _R)], sp.at[k])
    for h in range(_NBUF):
        pltpu.async_copy(x_hbm.at[pl.ds(h * _R, _R)], xb.at[h], sx.at[h])

    for g in range(n_blocks):
        s = g % _NBUF
        pltpu.make_async_copy(
            x_hbm.at[pl.ds(g * _R, _R)], xb.at[s], sx.at[s]).wait()
        if g < n_pos_chunks:
            pltpu.make_async_copy(
                p_hbm.at[pl.ds(g * _R, _R)], pv.at[pl.ds(g * _R, _R)],
                sp.at[g]).wait()
        if g >= _NBUF:
            pltpu.make_async_copy(
                ob.at[s], o_hbm.at[pl.ds((g - _NBUF) * _R, _R)], so.at[s]).wait()
        ob[s] = xb[s] + pv[pl.ds((g % n_pos_chunks) * _R, _R), :]
        pltpu.async_copy(ob.at[s], o_hbm.at[pl.ds(g * _R, _R)], so.at[s])
        h = g + _NBUF
        if h < n_blocks:
            pltpu.async_copy(x_hbm.at[pl.ds(h * _R, _R)], xb.at[s], sx.at[s])

    for t in range(n_blocks - _NBUF, n_blocks):
        s = t % _NBUF
        pltpu.make_async_copy(
            ob.at[s], o_hbm.at[pl.ds(t * _R, _R)], so.at[s]).wait()


def kernel(x, pos_emb):
    batch, seq_len, d_model = x.shape
    x2d = x.reshape(batch * seq_len, d_model)
    pe = pos_emb[:seq_len]
    out = pl.pallas_call(
        _ring_body,
        in_specs=[
            pl.BlockSpec(memory_space=pltpu.HBM),
            pl.BlockSpec(memory_space=pltpu.HBM),
        ],
        out_specs=pl.BlockSpec(memory_space=pltpu.HBM),
        out_shape=jax.ShapeDtypeStruct((batch * seq_len, d_model), x.dtype),
        scratch_shapes=[
            pltpu.VMEM((_NBUF, _R, d_model), jnp.float32),
            pltpu.VMEM((_NBUF, _R, d_model), jnp.float32),
            pltpu.VMEM((seq_len, d_model), jnp.float32),
            pltpu.SemaphoreType.DMA((_NBUF,)),
            pltpu.SemaphoreType.DMA((seq_len // _R,)),
            pltpu.SemaphoreType.DMA((_NBUF,)),
        ],
    )(x2d, pe)
    return out.reshape(batch, seq_len, d_model)


# speedup vs baseline: 1.0027x; 1.0027x over previous
import jax
import jax.numpy as jnp
from jax.experimental import pallas as pl
from jax.experimental.pallas import tpu as pltpu

_R = 2048
_NBUF = 3


def _ring_body(x_hbm, p_hbm, o_hbm, xb, ob, pv, sx, sp, so):
    n_rows = x_hbm.shape[0]
    seq_len = p_hbm.shape[0]
    n_blocks = n_rows // _R
    n_pos_chunks = seq_len // _R

    for k in range(n_pos_chunks):
        pltpu.async_copy(
            p_hbm.at[pl.ds(k * _R, _R)], pv.at[pl.ds(k * _R, _R)], sp.at[k])
    for h in range(_NBUF):
        pltpu.async_copy(x_hbm.at[pl.ds(h * _R, _R)], xb.at[h], sx.at[h])

    for g in range(n_blocks):
        s = g % _NBUF
        pltpu.make_async_copy(
            x_hbm.at[pl.ds(g * _R, _R)], xb.at[s], sx.at[s]).wait()
        if g < n_pos_chunks:
            pltpu.make_async_copy(
                p_hbm.at[pl.ds(g * _R, _R)], pv.at[pl.ds(g * _R, _R)],
                sp.at[g]).wait()
        if g >= _NBUF:
            pltpu.make_async_copy(
                ob.at[s], o_hbm.at[pl.ds((g - _NBUF) * _R, _R)], so.at[s]).wait()
        ob[s] = xb[s] + pv[pl.ds((g % n_pos_chunks) * _R, _R), :]
        pltpu.async_copy(ob.at[s], o_hbm.at[pl.ds(g * _R, _R)], so.at[s])
        h = g + _NBUF
        if h < n_blocks:
            pltpu.async_copy(x_hbm.at[pl.ds(h * _R, _R)], xb.at[s], sx.at[s])

    for t in range(n_blocks - _NBUF, n_blocks):
        s = t % _NBUF
        pltpu.make_async_copy(
            ob.at[s], o_hbm.at[pl.ds(t * _R, _R)], so.at[s]).wait()


def kernel(x, pos_emb):
    batch, seq_len, d_model = x.shape
    x2d = x.reshape(batch * seq_len, d_model)
    pe = pos_emb[:seq_len]
    out = pl.pallas_call(
        _ring_body,
        in_specs=[
            pl.BlockSpec(memory_space=pltpu.HBM),
            pl.BlockSpec(memory_space=pltpu.HBM),
        ],
        out_specs=pl.BlockSpec(memory_space=pltpu.HBM),
        out_shape=jax.ShapeDtypeStruct((batch * seq_len, d_model), x.dtype),
        scratch_shapes=[
            pltpu.VMEM((_NBUF, _R, d_model), jnp.float32),
            pltpu.VMEM((_NBUF, _R, d_model), jnp.float32),
            pltpu.VMEM((seq_len, d_model), jnp.float32),
            pltpu.SemaphoreType.DMA((_NBUF,)),
            pltpu.SemaphoreType.DMA((seq_len // _R,)),
            pltpu.SemaphoreType.DMA((_NBUF,)),
        ],
        compiler_params=pltpu.CompilerParams(
            vmem_limit_bytes=112 * 1024 * 1024,
        ),
    )(x2d, pe)
    return out.reshape(batch, seq_len, d_model)


# speedup vs baseline: 1.0033x; 1.0006x over previous
import jax
import jax.numpy as jnp
from jax.experimental import pallas as pl
from jax.experimental.pallas import tpu as pltpu

_R = 1024
_NBUF = 3


def _ring_body(x_hbm, p_hbm, o_hbm, xb, ob, pv, sx, sp, so):
    n_rows = x_hbm.shape[0]
    seq_len = p_hbm.shape[0]
    n_blocks = n_rows // _R
    n_pos_chunks = seq_len // _R

    for k in range(n_pos_chunks):
        pltpu.async_copy(
            p_hbm.at[pl.ds(k * _R, _R)], pv.at[pl.ds(k * _R, _R)], sp.at[k])
    for h in range(_NBUF):
        pltpu.async_copy(x_hbm.at[pl.ds(h * _R, _R)], xb.at[h], sx.at[h])

    for g in range(n_blocks):
        s = g % _NBUF
        pltpu.make_async_copy(
            x_hbm.at[pl.ds(g * _R, _R)], xb.at[s], sx.at[s]).wait()
        if g < n_pos_chunks:
            pltpu.make_async_copy(
                p_hbm.at[pl.ds(g * _R, _R)], pv.at[pl.ds(g * _R, _R)],
                sp.at[g]).wait()
        if g >= _NBUF:
            pltpu.make_async_copy(
                ob.at[s], o_hbm.at[pl.ds((g - _NBUF) * _R, _R)], so.at[s]).wait()
        ob[s] = xb[s] + pv[pl.ds((g % n_pos_chunks) * _R, _R), :]
        pltpu.async_copy(ob.at[s], o_hbm.at[pl.ds(g * _R, _R)], so.at[s])
        h = g + _NBUF
        if h < n_blocks:
            pltpu.async_copy(x_hbm.at[pl.ds(h * _R, _R)], xb.at[s], sx.at[s])

    for t in range(n_blocks - _NBUF, n_blocks):
        s = t % _NBUF
        pltpu.make_async_copy(
            ob.at[s], o_hbm.at[pl.ds(t * _R, _R)], so.at[s]).wait()


def kernel(x, pos_emb):
    batch, seq_len, d_model = x.shape
    x2d = x.reshape(batch * seq_len, d_model)
    pe = pos_emb[:seq_len]
    out = pl.pallas_call(
        _ring_body,
        in_specs=[
            pl.BlockSpec(memory_space=pltpu.HBM),
            pl.BlockSpec(memory_space=pltpu.HBM),
        ],
        out_specs=pl.BlockSpec(memory_space=pltpu.HBM),
        out_shape=jax.ShapeDtypeStruct((batch * seq_len, d_model), x.dtype),
        scratch_shapes=[
            pltpu.VMEM((_NBUF, _R, d_model), jnp.float32),
            pltpu.VMEM((_NBUF, _R, d_model), jnp.float32),
            pltpu.VMEM((seq_len, d_model), jnp.float32),
            pltpu.SemaphoreType.DMA((_NBUF,)),
            pltpu.SemaphoreType.DMA((seq_len // _R,)),
            pltpu.SemaphoreType.DMA((_NBUF,)),
        ],
    )(x2d, pe)
    return out.reshape(batch, seq_len, d_model)
